# baseline (speedup 1.0000x reference)
import jax
import jax.numpy as jnp
from jax.experimental import pallas as pl
from jax.experimental.pallas import tpu as pltpu

B = 64
S = 512
D = 2048
K = 8192
KT = 512
NKT = K // KT
RB = 2
P1 = B // RB
P2 = NKT
ALPHA_BASE = 0.3
MAX_DELTA = 0.5


NCACHE = 1


def _fused_kernel(h_ref, a_ref, o_ref, vraw_ref, vnorm_ref, best_ref, rmax_ref,
                  vdiff_ref, hcache_ref):
    i = pl.program_id(0)

    @pl.when(i < P1)
    def _phase_mean():
        m = jnp.mean(h_ref[...], axis=1)
        for r in range(RB):
            vraw_ref[pl.ds(i * RB + r, 1), :] = m[r][None, :]

    for c in range(NCACHE):
        @pl.when(i == P1 - 2 - c)
        def _stash(c=c):
            hcache_ref[c] = h_ref[...]

    @pl.when(jnp.logical_and(i >= P1, i < P1 + P2))
    def _phase_lookup():
        j = i - P1

        @pl.when(j == 0)
        def _init():
            v = vraw_ref[...]
            bm = jnp.mean(v, axis=0)
            bv = jnp.mean((v - bm[None, :]) ** 2, axis=0)
            vnorm_ref[...] = (v - bm[None, :]) / jnp.sqrt(bv + 1e-8)[None, :]
            rmax_ref[...] = jnp.full((B, 128), -jnp.inf, jnp.float32)
            best_ref[...] = jnp.zeros((B, D), jnp.float32)

        a = a_ref[...]
        rn = 1.0 / jnp.maximum(jnp.sqrt(jnp.sum(a * a, axis=1)), 1e-8)
        vn = vnorm_ref[...]
        cos = jax.lax.dot_general(
            vn, a, (((1,), (1,)), ((), ())),
            preferred_element_type=jnp.float32)
        cos = cos * rn[None, :]
        tile_max = jnp.max(cos, axis=1)
        tile_arg = jnp.argmax(cos, axis=1)
        run_max = rmax_ref[:, 0]
        improved = tile_max > run_max
        onehot = jnp.where(
            jax.lax.broadcasted_iota(jnp.int32, (B, KT), 1) == tile_arg[:, None],
            rn[None, :], 0.0)
        cand = jax.lax.dot_general(
            onehot, a, (((1,), (0,)), ((), ())),
            preferred_element_type=jnp.float32)
        best_ref[...] = jnp.where(improved[:, None], cand, best_ref[...])
        new_max = jnp.where(improved, tile_max, run_max)
        rmax_ref[...] = jnp.broadcast_to(new_max[:, None], (B, 128))

        @pl.when(j == P2 - 1)
        def _finish():
            vnorm = vnorm_ref[...]
            score = rmax_ref[:, 0]
            alpha = ALPHA_BASE * (1.0 - score)
            delta = jnp.clip(best_ref[...] - vnorm, -MAX_DELTA, MAX_DELTA)
            v_snapped = vnorm + alpha[:, None] * delta
            vdiff_ref[...] = v_snapped - vraw_ref[...]

    @pl.when(i >= P1 + P2)
    def _phase_add():
        b = (P1 - 1) - (i - (P1 + P2))
        rows = [vdiff_ref[pl.ds(b * RB + r, 1), :] for r in range(RB)]
        vd = jnp.concatenate(rows, axis=0)[:, None, :]

        @pl.when(jnp.logical_or(b == P1 - 1, b < P1 - 1 - NCACHE))
        def _from_stream():
            o_ref[...] = h_ref[...] + vd

        for c in range(NCACHE):
            @pl.when(b == P1 - 2 - c)
            def _from_cache(c=c):
                o_ref[...] = hcache_ref[c] + vd


def _h_index(i):
    b = jnp.where(i < P1, i,
                  jnp.where(i < P1 + P2, P1 - 1, (P1 - 1) - (i - (P1 + P2))))
    b = jnp.where(jnp.logical_and(i >= P1 + P2, b >= P1 - 1 - NCACHE),
                  P1 - 1, b)
    return (b, 0, 0)


def _a_index(i):
    j = jnp.clip(i - P1, 0, P2 - 1)
    return (j, 0)


def _o_index(i):
    b = jnp.where(i < P1 + P2, P1 - 1, (P1 - 1) - (i - (P1 + P2)))
    return (b, 0, 0)


@jax.jit
def kernel(hidden_states, attractors):
    return pl.pallas_call(
        _fused_kernel,
        grid=(P1 + P2 + P1,),
        in_specs=[
            pl.BlockSpec((RB, S, D), _h_index),
            pl.BlockSpec((KT, D), _a_index),
        ],
        out_specs=pl.BlockSpec((RB, S, D), _o_index),
        out_shape=jax.ShapeDtypeStruct((B, S, D), jnp.float32),
        scratch_shapes=[
            pltpu.VMEM((B, D), jnp.float32),
            pltpu.VMEM((B, D), jnp.float32),
            pltpu.VMEM((B, D), jnp.float32),
            pltpu.VMEM((B, 128), jnp.float32),
            pltpu.VMEM((B, D), jnp.float32),
            pltpu.VMEM((NCACHE, RB, S, D), jnp.float32),
        ],
    )(hidden_states, attractors)


# speedup vs baseline: 1.0100x; 1.0100x over previous
import jax
import jax.numpy as jnp
from jax.experimental import pallas as pl
from jax.experimental.pallas import tpu as pltpu

B = 64
S = 512
D = 2048
K = 8192
KT = 1024
NKT = K // KT
RB = 2
P1 = B // RB
P2 = NKT
ALPHA_BASE = 0.3
MAX_DELTA = 0.5


def _fused_kernel(h_ref, a_ref, o_ref, vraw_ref, vnorm_ref, best_ref, rmax_ref,
                  vdiff_ref):
    i = pl.program_id(0)

    @pl.when(i < P1)
    def _phase_mean():
        m = jnp.mean(h_ref[...], axis=1)
        for r in range(RB):
            vraw_ref[pl.ds(i * RB + r, 1), :] = m[r][None, :]

    @pl.when(jnp.logical_and(i >= P1, i < P1 + P2))
    def _phase_lookup():
        j = i - P1

        @pl.when(j == 0)
        def _init():
            v = vraw_ref[...]
            bm = jnp.mean(v, axis=0)
            bv = jnp.mean((v - bm[None, :]) ** 2, axis=0)
            vnorm_ref[...] = (v - bm[None, :]) / jnp.sqrt(bv + 1e-8)[None, :]
            rmax_ref[...] = jnp.full((B, 128), -jnp.inf, jnp.float32)
            best_ref[...] = jnp.zeros((B, D), jnp.float32)

        a = a_ref[...]
        rn = 1.0 / jnp.maximum(jnp.sqrt(jnp.sum(a * a, axis=1)), 1e-8)
        vn = vnorm_ref[...]
        cos = jax.lax.dot_general(
            vn, a, (((1,), (1,)), ((), ())),
            preferred_element_type=jnp.float32)
        cos = cos * rn[None, :]
        tile_max = jnp.max(cos, axis=1)
        tile_arg = jnp.argmax(cos, axis=1)
        run_max = rmax_ref[:, 0]
        improved = tile_max > run_max
        onehot = jnp.where(
            jax.lax.broadcasted_iota(jnp.int32, (B, KT), 1) == tile_arg[:, None],
            rn[None, :], 0.0)
        cand = jax.lax.dot_general(
            onehot, a, (((1,), (0,)), ((), ())),
            preferred_element_type=jnp.float32)
        best_ref[...] = jnp.where(improved[:, None], cand, best_ref[...])
        new_max = jnp.where(improved, tile_max, run_max)
        rmax_ref[...] = jnp.broadcast_to(new_max[:, None], (B, 128))

        @pl.when(j == P2 - 1)
        def _finish():
            vnorm = vnorm_ref[...]
            score = rmax_ref[:, 0]
            alpha = ALPHA_BASE * (1.0 - score)
            delta = jnp.clip(best_ref[...] - vnorm, -MAX_DELTA, MAX_DELTA)
            v_snapped = vnorm + alpha[:, None] * delta
            vdiff_ref[...] = v_snapped - vraw_ref[...]

    @pl.when(i >= P1 + P2)
    def _phase_add():
        b = (P1 - 1) - (i - (P1 + P2))
        rows = [vdiff_ref[pl.ds(b * RB + r, 1), :] for r in range(RB)]
        o_ref[...] = h_ref[...] + jnp.concatenate(rows, axis=0)[:, None, :]


def _h_index(i):
    b = jnp.where(i < P1, i,
                  jnp.where(i < P1 + P2, P1 - 1, (P1 - 1) - (i - (P1 + P2))))
    return (b, 0, 0)


def _a_index(i):
    j = jnp.clip(i - P1, 0, P2 - 1)
    return (j, 0)


def _o_index(i):
    b = jnp.where(i < P1 + P2, P1 - 1, (P1 - 1) - (i - (P1 + P2)))
    return (b, 0, 0)


@jax.jit
def kernel(hidden_states, attractors):
    return pl.pallas_call(
        _fused_kernel,
        grid=(P1 + P2 + P1,),
        in_specs=[
            pl.BlockSpec((RB, S, D), _h_index),
            pl.BlockSpec((KT, D), _a_index),
        ],
        out_specs=pl.BlockSpec((RB, S, D), _o_index),
        out_shape=jax.ShapeDtypeStruct((B, S, D), jnp.float32),
        scratch_shapes=[
            pltpu.VMEM((B, D), jnp.float32),
            pltpu.VMEM((B, D), jnp.float32),
            pltpu.VMEM((B, D), jnp.float32),
            pltpu.VMEM((B, 128), jnp.float32),
            pltpu.VMEM((B, D), jnp.float32),
        ],
    )(hidden_states, attractors)
